# Initial kernel scaffold; baseline (speedup 1.0000x reference)
import jax
import jax.numpy as jnp
from jax.experimental import pallas as pl


def kernel(x, W_enc, b_enc, W_dec, b_dec):
    raise NotImplementedError("write your pallas kernel here")


# speedup vs baseline: 4.0692x; 4.0692x over previous
import jax
import jax.numpy as jnp
from jax import lax
from jax.experimental import pallas as pl
from jax.experimental.pallas import tpu as pltpu

D_IN = 1024
D_LAT = 32768
TOPK = 64
BATCH = 64
TL_ENC = 2048
TL_DEC = 2048
N_ITER = 36


def _encode_body(x_ref, w_ref, b_ref, h_ref):
    acc = lax.dot_general(
        x_ref[...], w_ref[...], (((1,), (1,)), ((), ())),
        preferred_element_type=jnp.float32)
    h_ref[...] = jnp.maximum(acc + b_ref[...], 0.0)


def _thresh_body(h_ref, thr_ref):
    h = h_ref[...]
    m = jnp.max(h, axis=1, keepdims=True)
    lo = jnp.zeros_like(m)
    hi = m

    def body(_, carry):
        lo, hi = carry
        mid = 0.5 * (lo + hi)
        cnt = jnp.sum(jnp.where(h >= mid, 1.0, 0.0), axis=1, keepdims=True)
        ge = cnt >= TOPK
        return jnp.where(ge, mid, lo), jnp.where(ge, hi, mid)

    lo, hi = lax.fori_loop(0, N_ITER, body, (lo, hi))
    thr_ref[...] = jnp.broadcast_to(lo, thr_ref.shape)


def _decode_body(h_ref, thr_ref, w_ref, b_ref, htopk_ref, xhat_ref):
    j = pl.program_id(0)
    h = h_ref[...]
    thr = thr_ref[:, :1]
    ht = jnp.where(h >= thr, h, 0.0)
    htopk_ref[...] = ht
    part = lax.dot_general(
        ht, w_ref[...], (((1,), (1,)), ((), ())),
        preferred_element_type=jnp.float32)

    @pl.when(j == 0)
    def _():
        xhat_ref[...] = part + b_ref[...]

    @pl.when(j != 0)
    def _():
        xhat_ref[...] += part


def kernel(x, W_enc, b_enc, W_dec, b_dec):
    b_enc2 = b_enc.reshape(1, D_LAT)
    b_dec2 = b_dec.reshape(1, D_IN)

    h = pl.pallas_call(
        _encode_body,
        grid=(D_LAT // TL_ENC,),
        in_specs=[
            pl.BlockSpec((BATCH, D_IN), lambda j: (0, 0)),
            pl.BlockSpec((TL_ENC, D_IN), lambda j: (j, 0)),
            pl.BlockSpec((1, TL_ENC), lambda j: (0, j)),
        ],
        out_specs=pl.BlockSpec((BATCH, TL_ENC), lambda j: (0, j)),
        out_shape=jax.ShapeDtypeStruct((BATCH, D_LAT), jnp.float32),
    )(x, W_enc, b_enc2)

    thr = pl.pallas_call(
        _thresh_body,
        out_shape=jax.ShapeDtypeStruct((BATCH, 128), jnp.float32),
    )(h)

    h_topk, x_hat = pl.pallas_call(
        _decode_body,
        grid=(D_LAT // TL_DEC,),
        in_specs=[
            pl.BlockSpec((BATCH, TL_DEC), lambda j: (0, j)),
            pl.BlockSpec((BATCH, 128), lambda j: (0, 0)),
            pl.BlockSpec((D_IN, TL_DEC), lambda j: (0, j)),
            pl.BlockSpec((1, D_IN), lambda j: (0, 0)),
        ],
        out_specs=[
            pl.BlockSpec((BATCH, TL_DEC), lambda j: (0, j)),
            pl.BlockSpec((BATCH, D_IN), lambda j: (0, 0)),
        ],
        out_shape=[
            jax.ShapeDtypeStruct((BATCH, D_LAT), jnp.float32),
            jax.ShapeDtypeStruct((BATCH, D_IN), jnp.float32),
        ],
        compiler_params=pltpu.CompilerParams(
            dimension_semantics=("arbitrary",)),
    )(h, thr, W_dec, b_dec2)

    return (x_hat, h, h_topk)
